# initial kernel scaffold (unmeasured)
import jax
import jax.numpy as jnp
from jax import lax
from jax.experimental import pallas as pl
from jax.experimental.pallas import tpu as pltpu

N_DEV = 8
SQ = 1024
HQ = 8
DH = 128
NG = 4
GQ = SQ // NG
D_MODEL = 1024
SCALE = 0.08838834764831843


def _regroup(m):
    s = m.shape
    m = m.reshape(4, 4, 64, *s[1:])
    perm = (1, 0) + tuple(range(2, m.ndim))
    return m.transpose(*perm).reshape(s)


def kernel(x, Wq, K_ext, V_ext, Wo):
    assert x.shape == (1, SQ, D_MODEL), x.shape
    assert Wq.shape == (D_MODEL, HQ * DH), Wq.shape
    assert K_ext.shape == (N_DEV, SQ, N_DEV * HQ, DH), K_ext.shape
    assert Wo.shape == (HQ * DH, D_MODEL), Wo.shape

    def body(x_ref, wq_ref, k_hbm, v_hbm, wo_ref, out_ref,
             comm, stage, kbuf, vbuf,
             send_sems, recv_sems, stage_sem, k_sem, v_sem):
        my = lax.axis_index("i")
        right = lax.rem(my + 1, N_DEV)
        left = lax.rem(my + N_DEV - 1, N_DEV)

        stage[0] = wq_ref[...].astype(jnp.bfloat16)
        stage[1] = wo_ref[...].astype(jnp.bfloat16)
        cp = pltpu.make_async_copy(stage, comm.at[0], stage_sem)
        cp.start()
        cp.wait()

        xg = _regroup(x_ref[0].astype(jnp.bfloat16))

        rdma0 = pltpu.make_async_remote_copy(
            src_ref=comm.at[0], dst_ref=comm.at[1],
            send_sem=send_sems.at[0], recv_sem=recv_sems.at[0],
            device_id=(right,), device_id_type=pl.DeviceIdType.MESH,
        )
        rdma0.start()

        acc = jnp.zeros((SQ, D_MODEL), jnp.float32)
        for t in range(N_DEV):
            j = lax.rem(my + N_DEV - t, N_DEV)

            if t > 0:
                rdesc = pltpu.make_async_remote_copy(
                    src_ref=comm.at[t - 1], dst_ref=comm.at[t],
                    send_sem=send_sems.at[t - 1], recv_sem=recv_sems.at[t - 1],
                    device_id=(left,), device_id_type=pl.DeviceIdType.MESH,
                )
                rdesc.wait_recv()
                if t < N_DEV - 1:
                    fwd = pltpu.make_async_remote_copy(
                        src_ref=comm.at[t], dst_ref=comm.at[t + 1],
                        send_sem=send_sems.at[t], recv_sem=recv_sems.at[t],
                        device_id=(right,), device_id_type=pl.DeviceIdType.MESH,
                    )
                    fwd.start()
                scp = pltpu.make_async_copy(comm.at[t], stage, stage_sem)
                scp.start()

            kcp = pltpu.make_async_copy(
                k_hbm.at[my, :, pl.ds(j * HQ, HQ), :], kbuf, k_sem)
            vcp = pltpu.make_async_copy(
                v_hbm.at[my, :, pl.ds(j * HQ, HQ), :], vbuf, v_sem)
            kcp.start()
            vcp.start()
            if t > 0:
                scp.wait()
            kcp.wait()
            vcp.wait()

            wq = stage[0]
            wo = stage[1]

            q = jnp.dot(xg, wq, preferred_element_type=jnp.float32)
            q = (q * SCALE).astype(jnp.bfloat16)
            q = q.reshape(NG, GQ, HQ, DH).transpose(0, 2, 1, 3)

            k = _regroup(kbuf[...].astype(jnp.bfloat16))
            k = k.reshape(NG, GQ, HQ, DH).transpose(0, 2, 1, 3)
            v = _regroup(vbuf[...].astype(jnp.bfloat16))
            v = v.reshape(NG, GQ, HQ, DH).transpose(0, 2, 1, 3)

            ctx_groups = []
            for g in range(NG):
                sc = lax.dot_general(
                    q[g], k[g], (((2,), (2,)), ((0,), (0,))),
                    preferred_element_type=jnp.float32)
                m = jnp.max(sc, axis=-1, keepdims=True)
                e = jnp.exp(sc - m)
                w = (e / jnp.sum(e, axis=-1, keepdims=True)).astype(jnp.bfloat16)
                ctx = lax.dot_general(
                    w, v[g], (((2,), (1,)), ((0,), (0,))),
                    preferred_element_type=jnp.float32)
                ctx_groups.append(ctx.astype(jnp.bfloat16))
            ctx_all = jnp.stack(ctx_groups)
            ctx_all = ctx_all.transpose(0, 2, 1, 3).reshape(SQ, HQ * DH)
            acc = acc + jnp.dot(ctx_all, wo, preferred_element_type=jnp.float32)

        for s in range(N_DEV - 1):
            d = pltpu.make_async_remote_copy(
                src_ref=comm.at[s], dst_ref=comm.at[s + 1],
                send_sem=send_sems.at[s], recv_sem=recv_sems.at[s],
                device_id=(right,), device_id_type=pl.DeviceIdType.MESH,
            )
            d.wait_send()

        out_ref[0] = _regroup(acc)

    return pl.pallas_call(
        body,
        out_shape=jax.ShapeDtypeStruct((1, SQ, D_MODEL), jnp.float32),
        in_specs=[
            pl.BlockSpec(memory_space=pltpu.MemorySpace.VMEM),
            pl.BlockSpec(memory_space=pltpu.MemorySpace.VMEM),
            pl.BlockSpec(memory_space=pl.ANY),
            pl.BlockSpec(memory_space=pl.ANY),
            pl.BlockSpec(memory_space=pltpu.MemorySpace.VMEM),
        ],
        out_specs=pl.BlockSpec(memory_space=pltpu.MemorySpace.VMEM),
        scratch_shapes=[
            pltpu.MemorySpace.HBM((N_DEV, 2, D_MODEL, D_MODEL), jnp.bfloat16),
            pltpu.MemorySpace.VMEM((2, D_MODEL, D_MODEL), jnp.bfloat16),
            pltpu.MemorySpace.VMEM((SQ, HQ, DH), jnp.float32),
            pltpu.MemorySpace.VMEM((SQ, HQ, DH), jnp.float32),
            pltpu.SemaphoreType.DMA((N_DEV - 1,)),
            pltpu.SemaphoreType.DMA((N_DEV - 1,)),
            pltpu.SemaphoreType.DMA,
            pltpu.SemaphoreType.DMA,
            pltpu.SemaphoreType.DMA,
        ],
    )(x, Wq, K_ext, V_ext, Wo)


# baseline (device time: 378411 ns/iter reference)
import jax
import jax.numpy as jnp
from jax import lax
from jax.experimental import pallas as pl
from jax.experimental.pallas import tpu as pltpu

N_DEV = 8
SQ = 1024
HQ = 8
DH = 128
NG = 4
GQ = SQ // NG
D_MODEL = 1024
NSLOT = 3
SCALE = 0.08838834764831843


def _regroup(m):
    s = m.shape
    m = m.reshape(4, 4, 64, *s[1:])
    perm = (1, 0) + tuple(range(2, m.ndim))
    return m.transpose(*perm).reshape(s)


def kernel(x, Wq, K_ext, V_ext, Wo):
    assert x.shape == (1, SQ, D_MODEL), x.shape
    assert Wq.shape == (D_MODEL, HQ * DH), Wq.shape
    assert K_ext.shape == (N_DEV, SQ, N_DEV * HQ, DH), K_ext.shape
    assert Wo.shape == (HQ * DH, D_MODEL), Wo.shape

    def body(x_ref, wq_hbm, k_hbm, v_hbm, wo_hbm, out_ref,
             comm, wstage, kbuf, vbuf,
             send_sems, recv_sems, w_sem, k_sem, v_sem, credit_sem):
        my = lax.axis_index("i")
        right = lax.rem(my + 1, N_DEV)
        left = lax.rem(my + N_DEV - 1, N_DEV)

        cp = pltpu.make_async_copy(wq_hbm, wstage, w_sem)
        cp.start()
        cp.wait()
        comm[0, 0] = wstage[...].astype(jnp.bfloat16)
        cp = pltpu.make_async_copy(wo_hbm, wstage, w_sem)
        cp.start()
        cp.wait()
        comm[0, 1] = wstage[...].astype(jnp.bfloat16)

        xg = _regroup(x_ref[0].astype(jnp.bfloat16))

        out_ref[0, :, :] = jnp.zeros((SQ, D_MODEL), jnp.float32)

        def step(t, carry):
            slot = lax.rem(t, NSLOT)
            nslot = lax.rem(t + 1, NSLOT)
            pslot = lax.rem(t + NSLOT - 1, NSLOT)
            j = lax.rem(my + N_DEV - t, N_DEV)

            @pl.when(t > 0)
            def _():
                pltpu.make_async_remote_copy(
                    src_ref=comm.at[slot], dst_ref=comm.at[slot],
                    send_sem=send_sems.at[t - 1],
                    recv_sem=recv_sems.at[t - 1],
                    device_id=(left,), device_id_type=pl.DeviceIdType.MESH,
                ).wait_recv()

            @pl.when(t < N_DEV - 1)
            def _():
                @pl.when(t >= NSLOT - 1)
                def _():
                    pl.semaphore_wait(credit_sem, 1)
                pltpu.make_async_remote_copy(
                    src_ref=comm.at[slot], dst_ref=comm.at[nslot],
                    send_sem=send_sems.at[t], recv_sem=recv_sems.at[t],
                    device_id=(right,), device_id_type=pl.DeviceIdType.MESH,
                ).start()

            kcp = pltpu.make_async_copy(
                k_hbm.at[my, :, pl.ds(j * HQ, HQ), :], kbuf, k_sem)
            vcp = pltpu.make_async_copy(
                v_hbm.at[my, :, pl.ds(j * HQ, HQ), :], vbuf, v_sem)
            kcp.start()
            vcp.start()
            kcp.wait()
            vcp.wait()

            wq = comm[slot, 0]
            wo = comm[slot, 1]

            kk = kbuf[...].astype(jnp.bfloat16).reshape(4, 4, 64, HQ, DH)
            vv = vbuf[...].astype(jnp.bfloat16).reshape(4, 4, 64, HQ, DH)

            for g in range(NG):
                gs = g * GQ
                qg = jnp.dot(xg[gs:gs + GQ], wq,
                             preferred_element_type=jnp.float32)
                qg = (qg * SCALE).astype(jnp.bfloat16).reshape(GQ, HQ, DH)
                kg = kk[:, g].reshape(GQ, HQ, DH)
                vg = vv[:, g].reshape(GQ, HQ, DH)
                sc = lax.dot_general(
                    qg, kg, (((2,), (2,)), ((1,), (1,))),
                    preferred_element_type=jnp.float32)
                mx = jnp.max(sc, axis=-1, keepdims=True)
                e = jnp.exp(sc - mx)
                w = (e / jnp.sum(e, axis=-1, keepdims=True)).astype(jnp.bfloat16)
                ctx = lax.dot_general(
                    w, vg, (((2,), (0,)), ((0,), (1,))),
                    preferred_element_type=jnp.float32)
                ctx = ctx.transpose(1, 0, 2).reshape(GQ, HQ * DH)
                contrib = jnp.dot(ctx.astype(jnp.bfloat16), wo,
                                  preferred_element_type=jnp.float32)
                out_ref[0, gs:gs + GQ, :] = out_ref[0, gs:gs + GQ, :] + contrib

            @pl.when(t > 0)
            def _():
                pltpu.make_async_remote_copy(
                    src_ref=comm.at[pslot], dst_ref=comm.at[slot],
                    send_sem=send_sems.at[t - 1],
                    recv_sem=recv_sems.at[t - 1],
                    device_id=(right,), device_id_type=pl.DeviceIdType.MESH,
                ).wait_send()

                @pl.when(t - 1 <= N_DEV - 1 - NSLOT)
                def _():
                    pl.semaphore_signal(
                        credit_sem, inc=1,
                        device_id=(left,),
                        device_id_type=pl.DeviceIdType.MESH,
                    )

            return carry

        lax.fori_loop(0, N_DEV, step, None)

        out_ref[0] = _regroup(out_ref[0, :, :])

    return pl.pallas_call(
        body,
        out_shape=jax.ShapeDtypeStruct((1, SQ, D_MODEL), jnp.float32),
        in_specs=[
            pl.BlockSpec(memory_space=pltpu.MemorySpace.VMEM),
            pl.BlockSpec(memory_space=pl.ANY),
            pl.BlockSpec(memory_space=pl.ANY),
            pl.BlockSpec(memory_space=pl.ANY),
            pl.BlockSpec(memory_space=pl.ANY),
        ],
        out_specs=pl.BlockSpec(memory_space=pltpu.MemorySpace.VMEM),
        scratch_shapes=[
            pltpu.MemorySpace.VMEM((NSLOT, 2, D_MODEL, D_MODEL), jnp.bfloat16),
            pltpu.MemorySpace.VMEM((D_MODEL, D_MODEL), jnp.float32),
            pltpu.MemorySpace.VMEM((SQ, HQ, DH), jnp.float32),
            pltpu.MemorySpace.VMEM((SQ, HQ, DH), jnp.float32),
            pltpu.SemaphoreType.DMA((N_DEV - 1,)),
            pltpu.SemaphoreType.DMA((N_DEV - 1,)),
            pltpu.SemaphoreType.DMA,
            pltpu.SemaphoreType.DMA,
            pltpu.SemaphoreType.DMA,
            pltpu.SemaphoreType.REGULAR,
        ],
        compiler_params=pltpu.CompilerParams(
            vmem_limit_bytes=64 * 1024 * 1024,
        ),
    )(x, Wq, K_ext, V_ext, Wo)


# device time: 306983 ns/iter; 1.2327x vs baseline; 1.2327x over previous
import jax
import jax.numpy as jnp
from jax import lax
from jax.experimental import pallas as pl
from jax.experimental.pallas import tpu as pltpu

N_DEV = 8
SQ = 1024
HQ = 8
DH = 128
NG = 4
GQ = SQ // NG
NB = 4
D_MODEL = 1024
NSLOT = 3
R_HOPS = 4
L_HOPS = 3
N_STEPS = 5
SCALE = 0.08838834764831843


def _regroup(m):
    s = m.shape
    m = m.reshape(4, 4, 64, *s[1:])
    perm = (1, 0) + tuple(range(2, m.ndim))
    return m.transpose(*perm).reshape(s)


def kernel(x, Wq, K_ext, V_ext, Wo):
    assert x.shape == (1, SQ, D_MODEL), x.shape
    assert Wq.shape == (D_MODEL, HQ * DH), Wq.shape
    assert K_ext.shape == (N_DEV, SQ, N_DEV * HQ, DH), K_ext.shape
    assert Wo.shape == (HQ * DH, D_MODEL), Wo.shape

    def body(x_ref, wq_hbm, k_hbm, v_hbm, wo_hbm, out_ref,
             commr, comml, wstage, xg_ref, kbuf, vbuf,
             r_send, r_recv, l_send, l_recv,
             w_sem, k_sems, v_sems, credit_sem):
        my = lax.axis_index("i")
        right = lax.rem(my + 1, N_DEV)
        left = lax.rem(my + N_DEV - 1, N_DEV)

        cp = pltpu.make_async_copy(wq_hbm, wstage, w_sem)
        cp.start()
        cp.wait()
        commr[0, 0] = wstage[...].astype(jnp.bfloat16)
        cp = pltpu.make_async_copy(wo_hbm, wstage, w_sem)
        cp.start()
        cp.wait()
        commr[0, 1] = wstage[...].astype(jnp.bfloat16)

        xg_ref[...] = _regroup(x_ref[0].astype(jnp.bfloat16))

        out_ref[0, :, :] = jnp.zeros((SQ, D_MODEL), jnp.float32)

        def consume(j, comm_ref, slot):
            wq = comm_ref[slot, 0]
            wo = comm_ref[slot, 1]

            def grp(g, carry):
                for b in range(NB):
                    pltpu.make_async_copy(
                        k_hbm.at[my, pl.ds((g + 4 * b) * 64, 64),
                                 pl.ds(j * HQ, HQ), :],
                        kbuf.at[pl.ds(64 * b, 64)], k_sems.at[b]).start()
                    pltpu.make_async_copy(
                        v_hbm.at[my, pl.ds((g + 4 * b) * 64, 64),
                                 pl.ds(j * HQ, HQ), :],
                        vbuf.at[pl.ds(64 * b, 64)], v_sems.at[b]).start()
                for b in range(NB):
                    pltpu.make_async_copy(
                        k_hbm.at[my, pl.ds((g + 4 * b) * 64, 64),
                                 pl.ds(j * HQ, HQ), :],
                        kbuf.at[pl.ds(64 * b, 64)], k_sems.at[b]).wait()
                    pltpu.make_async_copy(
                        v_hbm.at[my, pl.ds((g + 4 * b) * 64, 64),
                                 pl.ds(j * HQ, HQ), :],
                        vbuf.at[pl.ds(64 * b, 64)], v_sems.at[b]).wait()

                qg = jnp.dot(
                    xg_ref[pl.ds(g * GQ, GQ), :], wq,
                    preferred_element_type=jnp.float32)
                qg = (qg * SCALE).astype(jnp.bfloat16).reshape(GQ, HQ, DH)
                kg = kbuf[...].astype(jnp.bfloat16)
                vg = vbuf[...].astype(jnp.bfloat16)
                sc = lax.dot_general(
                    qg, kg, (((2,), (2,)), ((1,), (1,))),
                    preferred_element_type=jnp.float32)
                mx = jnp.max(sc, axis=-1, keepdims=True)
                e = jnp.exp(sc - mx)
                w = (e / jnp.sum(e, axis=-1, keepdims=True)).astype(jnp.bfloat16)
                ctx = lax.dot_general(
                    w, vg, (((2,), (0,)), ((0,), (1,))),
                    preferred_element_type=jnp.float32)
                ctx = ctx.transpose(1, 0, 2).reshape(GQ, HQ * DH)
                contrib = jnp.dot(ctx.astype(jnp.bfloat16), wo,
                                  preferred_element_type=jnp.float32)
                cur = out_ref[0, pl.ds(g * GQ, GQ), :]
                out_ref[0, pl.ds(g * GQ, GQ), :] = cur + contrib
                return carry

            lax.fori_loop(0, NG, grp, None)

        def r_fwd(s, slot, nslot):
            return pltpu.make_async_remote_copy(
                src_ref=commr.at[slot], dst_ref=commr.at[nslot],
                send_sem=r_send.at[s], recv_sem=r_recv.at[s],
                device_id=(right,), device_id_type=pl.DeviceIdType.MESH,
            )

        def l_fwd(s, src, nslot):
            return pltpu.make_async_remote_copy(
                src_ref=src, dst_ref=comml.at[nslot],
                send_sem=l_send.at[s], recv_sem=l_recv.at[s],
                device_id=(left,), device_id_type=pl.DeviceIdType.MESH,
            )

        def step(s, carry):
            slot = lax.rem(s, NSLOT)
            nslot = lax.rem(s + 1, NSLOT)
            pslot = lax.rem(s + NSLOT - 1, NSLOT)

            @pl.when(s >= 1)
            def _():
                pltpu.make_async_remote_copy(
                    src_ref=commr.at[slot], dst_ref=commr.at[slot],
                    send_sem=r_send.at[s - 1], recv_sem=r_recv.at[s - 1],
                    device_id=(left,), device_id_type=pl.DeviceIdType.MESH,
                ).wait_recv()

            @pl.when(jnp.logical_and(s >= 1, s <= L_HOPS))
            def _():
                pltpu.make_async_remote_copy(
                    src_ref=comml.at[slot], dst_ref=comml.at[slot],
                    send_sem=l_send.at[s - 1], recv_sem=l_recv.at[s - 1],
                    device_id=(right,), device_id_type=pl.DeviceIdType.MESH,
                ).wait_recv()

            @pl.when(s <= R_HOPS - 1)
            def _():
                @pl.when(s >= 2)
                def _():
                    pl.semaphore_wait(credit_sem, 1)
                r_fwd(s, slot, nslot).start()

            @pl.when(s == 0)
            def _():
                l_fwd(0, commr.at[0], 1).start()

            @pl.when(jnp.logical_and(s >= 1, s <= L_HOPS - 1))
            def _():
                l_fwd(s, comml.at[slot], nslot).start()

            consume(lax.rem(my + N_DEV - s, N_DEV), commr, slot)

            @pl.when(jnp.logical_and(s >= 1, s <= L_HOPS))
            def _():
                consume(lax.rem(my + s, N_DEV), comml, slot)

            @pl.when(s >= 1)
            def _():
                r_fwd(s - 1, pslot, slot).wait_send()

                @pl.when(s <= L_HOPS)
                def _():
                    l_fwd(s - 1, commr.at[0], slot).wait_send()

                @pl.when(s <= 2)
                def _():
                    pl.semaphore_signal(
                        credit_sem, inc=1,
                        device_id=(left,),
                        device_id_type=pl.DeviceIdType.MESH,
                    )

            return carry

        lax.fori_loop(0, N_STEPS, step, None)

        out_ref[0] = _regroup(out_ref[0, :, :])

    return pl.pallas_call(
        body,
        out_shape=jax.ShapeDtypeStruct((1, SQ, D_MODEL), jnp.float32),
        in_specs=[
            pl.BlockSpec(memory_space=pltpu.MemorySpace.VMEM),
            pl.BlockSpec(memory_space=pl.ANY),
            pl.BlockSpec(memory_space=pl.ANY),
            pl.BlockSpec(memory_space=pl.ANY),
            pl.BlockSpec(memory_space=pl.ANY),
        ],
        out_specs=pl.BlockSpec(memory_space=pltpu.MemorySpace.VMEM),
        scratch_shapes=[
            pltpu.MemorySpace.VMEM((NSLOT, 2, D_MODEL, D_MODEL), jnp.bfloat16),
            pltpu.MemorySpace.VMEM((NSLOT, 2, D_MODEL, D_MODEL), jnp.bfloat16),
            pltpu.MemorySpace.VMEM((D_MODEL, D_MODEL), jnp.float32),
            pltpu.MemorySpace.VMEM((SQ, D_MODEL), jnp.bfloat16),
            pltpu.MemorySpace.VMEM((GQ, HQ, DH), jnp.float32),
            pltpu.MemorySpace.VMEM((GQ, HQ, DH), jnp.float32),
            pltpu.SemaphoreType.DMA((R_HOPS,)),
            pltpu.SemaphoreType.DMA((R_HOPS,)),
            pltpu.SemaphoreType.DMA((L_HOPS,)),
            pltpu.SemaphoreType.DMA((L_HOPS,)),
            pltpu.SemaphoreType.DMA,
            pltpu.SemaphoreType.DMA((NB,)),
            pltpu.SemaphoreType.DMA((NB,)),
            pltpu.SemaphoreType.REGULAR,
        ],
        compiler_params=pltpu.CompilerParams(
            vmem_limit_bytes=64 * 1024 * 1024,
        ),
    )(x, Wq, K_ext, V_ext, Wo)


# device time: 274938 ns/iter; 1.3764x vs baseline; 1.1166x over previous
import jax
import jax.numpy as jnp
from jax import lax
from jax.experimental import pallas as pl
from jax.experimental.pallas import tpu as pltpu

N_DEV = 8
SQ = 1024
HQ = 8
DH = 128
NG = 4
GQ = SQ // NG
NB = 4
D_MODEL = 1024
NSLOT = 3
R_HOPS = 4
L_HOPS = 3
N_STEPS = 5
SCALE = 0.08838834764831843


def _regroup(m):
    s = m.shape
    m = m.reshape(4, 4, 64, *s[1:])
    perm = (1, 0) + tuple(range(2, m.ndim))
    return m.transpose(*perm).reshape(s)


def kernel(x, Wq, K_ext, V_ext, Wo):
    assert x.shape == (1, SQ, D_MODEL), x.shape
    assert Wq.shape == (D_MODEL, HQ * DH), Wq.shape
    assert K_ext.shape == (N_DEV, SQ, N_DEV * HQ, DH), K_ext.shape
    assert Wo.shape == (HQ * DH, D_MODEL), Wo.shape

    def body(x_ref, wq_hbm, k_hbm, v_hbm, wo_hbm, out_ref,
             commr, comml, wstage, xg_ref, kbuf, vbuf,
             r_send, r_recv, l_send, l_recv,
             w_sem, k_sems, v_sems, credit_sem):
        my = lax.axis_index("i")
        right = lax.rem(my + 1, N_DEV)
        left = lax.rem(my + N_DEV - 1, N_DEV)

        cp = pltpu.make_async_copy(wq_hbm, wstage, w_sem)
        cp.start()
        cp.wait()
        commr[0, 0] = wstage[...].astype(jnp.bfloat16)
        cp = pltpu.make_async_copy(wo_hbm, wstage, w_sem)
        cp.start()
        cp.wait()
        commr[0, 1] = wstage[...].astype(jnp.bfloat16)

        xg_ref[...] = _regroup(x_ref[0].astype(jnp.bfloat16))

        out_ref[0, :, :] = jnp.zeros((SQ, D_MODEL), jnp.float32)

        def kv_fetch(j, g, buf):
            for b in range(NB):
                pltpu.make_async_copy(
                    k_hbm.at[my, pl.ds((g + 4 * b) * 64, 64),
                             pl.ds(j * HQ, HQ), :],
                    kbuf.at[buf, pl.ds(64 * b, 64)],
                    k_sems.at[buf, b]).start()
                pltpu.make_async_copy(
                    v_hbm.at[my, pl.ds((g + 4 * b) * 64, 64),
                             pl.ds(j * HQ, HQ), :],
                    vbuf.at[buf, pl.ds(64 * b, 64)],
                    v_sems.at[buf, b]).start()

        def kv_wait(j, g, buf):
            for b in range(NB):
                pltpu.make_async_copy(
                    k_hbm.at[my, pl.ds((g + 4 * b) * 64, 64),
                             pl.ds(j * HQ, HQ), :],
                    kbuf.at[buf, pl.ds(64 * b, 64)],
                    k_sems.at[buf, b]).wait()
                pltpu.make_async_copy(
                    v_hbm.at[my, pl.ds((g + 4 * b) * 64, 64),
                             pl.ds(j * HQ, HQ), :],
                    vbuf.at[buf, pl.ds(64 * b, 64)],
                    v_sems.at[buf, b]).wait()

        def consume(j, comm_ref, slot):
            wq = comm_ref[slot, 0]
            wo = comm_ref[slot, 1]

            kv_fetch(j, 0, 0)

            def grp(g, carry):
                buf = lax.rem(g, 2)
                @pl.when(g < NG - 1)
                def _():
                    kv_fetch(j, g + 1, lax.rem(g + 1, 2))
                kv_wait(j, g, buf)

                qg = jnp.dot(
                    xg_ref[pl.ds(g * GQ, GQ), :], wq,
                    preferred_element_type=jnp.float32)
                qg = (qg * SCALE).astype(jnp.bfloat16).reshape(GQ, HQ, DH)
                kg = kbuf[buf].astype(jnp.bfloat16)
                vg = vbuf[buf].astype(jnp.bfloat16)
                sc = lax.dot_general(
                    qg, kg, (((2,), (2,)), ((1,), (1,))),
                    preferred_element_type=jnp.float32)
                mx = jnp.max(sc, axis=-1, keepdims=True)
                e = jnp.exp(sc - mx)
                w = (e / jnp.sum(e, axis=-1, keepdims=True)).astype(jnp.bfloat16)
                ctx = lax.dot_general(
                    w, vg, (((2,), (0,)), ((0,), (1,))),
                    preferred_element_type=jnp.float32)
                ctx = ctx.transpose(1, 0, 2).reshape(GQ, HQ * DH)
                contrib = jnp.dot(ctx.astype(jnp.bfloat16), wo,
                                  preferred_element_type=jnp.float32)
                cur = out_ref[0, pl.ds(g * GQ, GQ), :]
                out_ref[0, pl.ds(g * GQ, GQ), :] = cur + contrib
                return carry

            lax.fori_loop(0, NG, grp, None)

        def r_fwd(s, slot, nslot):
            return pltpu.make_async_remote_copy(
                src_ref=commr.at[slot], dst_ref=commr.at[nslot],
                send_sem=r_send.at[s], recv_sem=r_recv.at[s],
                device_id=(right,), device_id_type=pl.DeviceIdType.MESH,
            )

        def l_fwd(s, src, nslot):
            return pltpu.make_async_remote_copy(
                src_ref=src, dst_ref=comml.at[nslot],
                send_sem=l_send.at[s], recv_sem=l_recv.at[s],
                device_id=(left,), device_id_type=pl.DeviceIdType.MESH,
            )

        def step(s, carry):
            slot = lax.rem(s, NSLOT)
            nslot = lax.rem(s + 1, NSLOT)
            pslot = lax.rem(s + NSLOT - 1, NSLOT)

            @pl.when(s >= 1)
            def _():
                pltpu.make_async_remote_copy(
                    src_ref=commr.at[slot], dst_ref=commr.at[slot],
                    send_sem=r_send.at[s - 1], recv_sem=r_recv.at[s - 1],
                    device_id=(left,), device_id_type=pl.DeviceIdType.MESH,
                ).wait_recv()

            @pl.when(s <= R_HOPS - 1)
            def _():
                @pl.when(s >= 2)
                def _():
                    pl.semaphore_wait(credit_sem, 1)
                r_fwd(s, slot, nslot).start()

            @pl.when(s == 0)
            def _():
                l_fwd(0, commr.at[0], 1).start()

            consume(lax.rem(my + N_DEV - s, N_DEV), commr, slot)

            @pl.when(jnp.logical_and(s >= 1, s <= L_HOPS))
            def _():
                pltpu.make_async_remote_copy(
                    src_ref=comml.at[slot], dst_ref=comml.at[slot],
                    send_sem=l_send.at[s - 1], recv_sem=l_recv.at[s - 1],
                    device_id=(right,), device_id_type=pl.DeviceIdType.MESH,
                ).wait_recv()

                @pl.when(s <= L_HOPS - 1)
                def _():
                    l_fwd(s, comml.at[slot], nslot).start()

                consume(lax.rem(my + s, N_DEV), comml, slot)

            @pl.when(s >= 1)
            def _():
                r_fwd(s - 1, pslot, slot).wait_send()

                @pl.when(s <= L_HOPS)
                def _():
                    l_fwd(s - 1, commr.at[0], slot).wait_send()

                @pl.when(s <= 2)
                def _():
                    pl.semaphore_signal(
                        credit_sem, inc=1,
                        device_id=(left,),
                        device_id_type=pl.DeviceIdType.MESH,
                    )

            return carry

        lax.fori_loop(0, N_STEPS, step, None)

        out_ref[0] = _regroup(out_ref[0, :, :])

    return pl.pallas_call(
        body,
        out_shape=jax.ShapeDtypeStruct((1, SQ, D_MODEL), jnp.float32),
        in_specs=[
            pl.BlockSpec(memory_space=pltpu.MemorySpace.VMEM),
            pl.BlockSpec(memory_space=pl.ANY),
            pl.BlockSpec(memory_space=pl.ANY),
            pl.BlockSpec(memory_space=pl.ANY),
            pl.BlockSpec(memory_space=pl.ANY),
        ],
        out_specs=pl.BlockSpec(memory_space=pltpu.MemorySpace.VMEM),
        scratch_shapes=[
            pltpu.MemorySpace.VMEM((NSLOT, 2, D_MODEL, D_MODEL), jnp.bfloat16),
            pltpu.MemorySpace.VMEM((NSLOT, 2, D_MODEL, D_MODEL), jnp.bfloat16),
            pltpu.MemorySpace.VMEM((D_MODEL, D_MODEL), jnp.float32),
            pltpu.MemorySpace.VMEM((SQ, D_MODEL), jnp.bfloat16),
            pltpu.MemorySpace.VMEM((2, GQ, HQ, DH), jnp.float32),
            pltpu.MemorySpace.VMEM((2, GQ, HQ, DH), jnp.float32),
            pltpu.SemaphoreType.DMA((R_HOPS,)),
            pltpu.SemaphoreType.DMA((R_HOPS,)),
            pltpu.SemaphoreType.DMA((L_HOPS,)),
            pltpu.SemaphoreType.DMA((L_HOPS,)),
            pltpu.SemaphoreType.DMA,
            pltpu.SemaphoreType.DMA((2, NB)),
            pltpu.SemaphoreType.DMA((2, NB)),
            pltpu.SemaphoreType.REGULAR,
        ],
        compiler_params=pltpu.CompilerParams(
            vmem_limit_bytes=64 * 1024 * 1024,
        ),
    )(x, Wq, K_ext, V_ext, Wo)


# device time: 269621 ns/iter; 1.4035x vs baseline; 1.0197x over previous
import jax
import jax.numpy as jnp
from jax import lax
from jax.experimental import pallas as pl
from jax.experimental.pallas import tpu as pltpu

N_DEV = 8
SQ = 1024
HQ = 8
DH = 128
NG = 4
GQ = SQ // NG
NB = 4
D_MODEL = 1024
NSLOT = 3
R_HOPS = 4
L_HOPS = 3
N_STEPS = 5
SCALE = 0.08838834764831843


def _regroup(m):
    s = m.shape
    m = m.reshape(4, 4, 64, *s[1:])
    perm = (1, 0) + tuple(range(2, m.ndim))
    return m.transpose(*perm).reshape(s)


def kernel(x, Wq, K_ext, V_ext, Wo):
    assert x.shape == (1, SQ, D_MODEL), x.shape
    assert Wq.shape == (D_MODEL, HQ * DH), Wq.shape
    assert K_ext.shape == (N_DEV, SQ, N_DEV * HQ, DH), K_ext.shape
    assert Wo.shape == (HQ * DH, D_MODEL), Wo.shape

    def body(x_ref, wq_hbm, k_hbm, v_hbm, wo_hbm, out_ref,
             commr, comml, wstage, xg_ref, kbuf, vbuf,
             r_send, r_recv, l_send, l_recv,
             w_sem, k_sems, v_sems, credit_sem):
        my = lax.axis_index("i")
        right = lax.rem(my + 1, N_DEV)
        left = lax.rem(my + N_DEV - 1, N_DEV)

        cp = pltpu.make_async_copy(wq_hbm, wstage, w_sem)
        cp.start()
        cp.wait()
        commr[0, 0] = wstage[...].astype(jnp.bfloat16)
        cp = pltpu.make_async_copy(wo_hbm, wstage, w_sem)
        cp.start()
        cp.wait()
        commr[0, 1] = wstage[...].astype(jnp.bfloat16)

        xg_ref[...] = _regroup(x_ref[0].astype(jnp.bfloat16))

        out_ref[0, :, :] = jnp.zeros((SQ, D_MODEL), jnp.float32)

        def kv_fetch(j, g, buf):
            for b in range(NB):
                pltpu.make_async_copy(
                    k_hbm.at[my, pl.ds((g + 4 * b) * 64, 64),
                             pl.ds(j * HQ, HQ), :],
                    kbuf.at[buf, pl.ds(64 * b, 64)],
                    k_sems.at[buf, b]).start()
                pltpu.make_async_copy(
                    v_hbm.at[my, pl.ds((g + 4 * b) * 64, 64),
                             pl.ds(j * HQ, HQ), :],
                    vbuf.at[buf, pl.ds(64 * b, 64)],
                    v_sems.at[buf, b]).start()

        def kv_wait(j, g, buf):
            for b in range(NB):
                pltpu.make_async_copy(
                    k_hbm.at[my, pl.ds((g + 4 * b) * 64, 64),
                             pl.ds(j * HQ, HQ), :],
                    kbuf.at[buf, pl.ds(64 * b, 64)],
                    k_sems.at[buf, b]).wait()
                pltpu.make_async_copy(
                    v_hbm.at[my, pl.ds((g + 4 * b) * 64, 64),
                             pl.ds(j * HQ, HQ), :],
                    vbuf.at[buf, pl.ds(64 * b, 64)],
                    v_sems.at[buf, b]).wait()

        def consume(j, comm_ref, slot):
            wq = comm_ref[slot, 0]
            wo = comm_ref[slot, 1]

            kv_fetch(j, 0, 0)

            def grp(g, carry):
                buf = lax.rem(g, 2)
                @pl.when(g < NG - 1)
                def _():
                    kv_fetch(j, g + 1, lax.rem(g + 1, 2))
                kv_wait(j, g, buf)

                qg = jnp.dot(
                    xg_ref[pl.ds(g * GQ, GQ), :], wq,
                    preferred_element_type=jnp.float32)
                qg = (qg * SCALE).astype(jnp.bfloat16).reshape(GQ, HQ, DH)
                kg = kbuf[buf].astype(jnp.bfloat16)
                vg = vbuf[buf].astype(jnp.bfloat16)
                sc = lax.dot_general(
                    qg, kg, (((2,), (2,)), ((1,), (1,))),
                    preferred_element_type=jnp.float32)
                e = jnp.exp(sc)
                denom = jnp.sum(e, axis=-1, keepdims=True)
                ctx = lax.dot_general(
                    e.astype(jnp.bfloat16), vg,
                    (((2,), (0,)), ((0,), (1,))),
                    preferred_element_type=jnp.float32)
                ctx = ctx * (1.0 / denom)
                ctx = ctx.transpose(1, 0, 2).reshape(GQ, HQ * DH)
                contrib = jnp.dot(ctx.astype(jnp.bfloat16), wo,
                                  preferred_element_type=jnp.float32)
                cur = out_ref[0, pl.ds(g * GQ, GQ), :]
                out_ref[0, pl.ds(g * GQ, GQ), :] = cur + contrib
                return carry

            lax.fori_loop(0, NG, grp, None)

        def r_fwd(s, slot, nslot):
            return pltpu.make_async_remote_copy(
                src_ref=commr.at[slot], dst_ref=commr.at[nslot],
                send_sem=r_send.at[s], recv_sem=r_recv.at[s],
                device_id=(right,), device_id_type=pl.DeviceIdType.MESH,
            )

        def l_fwd(s, src, nslot):
            return pltpu.make_async_remote_copy(
                src_ref=src, dst_ref=comml.at[nslot],
                send_sem=l_send.at[s], recv_sem=l_recv.at[s],
                device_id=(left,), device_id_type=pl.DeviceIdType.MESH,
            )

        def step(s, carry):
            slot = lax.rem(s, NSLOT)
            nslot = lax.rem(s + 1, NSLOT)
            pslot = lax.rem(s + NSLOT - 1, NSLOT)

            @pl.when(s >= 1)
            def _():
                pltpu.make_async_remote_copy(
                    src_ref=commr.at[slot], dst_ref=commr.at[slot],
                    send_sem=r_send.at[s - 1], recv_sem=r_recv.at[s - 1],
                    device_id=(left,), device_id_type=pl.DeviceIdType.MESH,
                ).wait_recv()

            @pl.when(s <= R_HOPS - 1)
            def _():
                @pl.when(s >= 2)
                def _():
                    pl.semaphore_wait(credit_sem, 1)
                r_fwd(s, slot, nslot).start()

            @pl.when(s == 0)
            def _():
                l_fwd(0, commr.at[0], 1).start()

            consume(lax.rem(my + N_DEV - s, N_DEV), commr, slot)

            @pl.when(jnp.logical_and(s >= 1, s <= L_HOPS))
            def _():
                pltpu.make_async_remote_copy(
                    src_ref=comml.at[slot], dst_ref=comml.at[slot],
                    send_sem=l_send.at[s - 1], recv_sem=l_recv.at[s - 1],
                    device_id=(right,), device_id_type=pl.DeviceIdType.MESH,
                ).wait_recv()

                @pl.when(s <= L_HOPS - 1)
                def _():
                    l_fwd(s, comml.at[slot], nslot).start()

                consume(lax.rem(my + s, N_DEV), comml, slot)

            @pl.when(s >= 1)
            def _():
                r_fwd(s - 1, pslot, slot).wait_send()

                @pl.when(s <= L_HOPS)
                def _():
                    l_fwd(s - 1, commr.at[0], slot).wait_send()

                @pl.when(s <= 2)
                def _():
                    pl.semaphore_signal(
                        credit_sem, inc=1,
                        device_id=(left,),
                        device_id_type=pl.DeviceIdType.MESH,
                    )

            return carry

        lax.fori_loop(0, N_STEPS, step, None)

        out_ref[0] = _regroup(out_ref[0, :, :])

    return pl.pallas_call(
        body,
        out_shape=jax.ShapeDtypeStruct((1, SQ, D_MODEL), jnp.float32),
        in_specs=[
            pl.BlockSpec(memory_space=pltpu.MemorySpace.VMEM),
            pl.BlockSpec(memory_space=pl.ANY),
            pl.BlockSpec(memory_space=pl.ANY),
            pl.BlockSpec(memory_space=pl.ANY),
            pl.BlockSpec(memory_space=pl.ANY),
        ],
        out_specs=pl.BlockSpec(memory_space=pltpu.MemorySpace.VMEM),
        scratch_shapes=[
            pltpu.MemorySpace.VMEM((NSLOT, 2, D_MODEL, D_MODEL), jnp.bfloat16),
            pltpu.MemorySpace.VMEM((NSLOT, 2, D_MODEL, D_MODEL), jnp.bfloat16),
            pltpu.MemorySpace.VMEM((D_MODEL, D_MODEL), jnp.float32),
            pltpu.MemorySpace.VMEM((SQ, D_MODEL), jnp.bfloat16),
            pltpu.MemorySpace.VMEM((2, GQ, HQ, DH), jnp.float32),
            pltpu.MemorySpace.VMEM((2, GQ, HQ, DH), jnp.float32),
            pltpu.SemaphoreType.DMA((R_HOPS,)),
            pltpu.SemaphoreType.DMA((R_HOPS,)),
            pltpu.SemaphoreType.DMA((L_HOPS,)),
            pltpu.SemaphoreType.DMA((L_HOPS,)),
            pltpu.SemaphoreType.DMA,
            pltpu.SemaphoreType.DMA((2, NB)),
            pltpu.SemaphoreType.DMA((2, NB)),
            pltpu.SemaphoreType.REGULAR,
        ],
        compiler_params=pltpu.CompilerParams(
            vmem_limit_bytes=64 * 1024 * 1024,
        ),
    )(x, Wq, K_ext, V_ext, Wo)
